# Initial kernel scaffold; baseline (speedup 1.0000x reference)
import jax
import jax.numpy as jnp
from jax.experimental import pallas as pl


def kernel(rpn_cls_prob, rpn_bbox_pred, im_info, anchors):
    raise NotImplementedError("write your pallas kernel here")


# speedup vs baseline: 2.4970x; 2.4970x over previous
import math

import jax
import jax.numpy as jnp
from jax import lax
from jax.experimental import pallas as pl
from jax.experimental.pallas import tpu as pltpu

_A = 15
_H = 128
_W = 128
_HW = _H * _W
_N_TOT = _A * _HW
_PRE = 6000
_POST = 1000
_THRESH = 0.7
_STRIDE = 16.0
_CLIP = float(math.log(1000.0 / 16.0))
_MIN_SIZE = 0.0
_NW = 32
_HW_PER_W = _HW // _NW
_ROWS = 47
_PAD = _ROWS * 128


def _prep_body(sc_ref, dx_ref, dy_ref, dw_ref, dh_ref,
               wa_ref, ha_ref, cx_ref, cy_ref, im_ref,
               out_ref, offs_ref):
    f32 = jnp.float32
    score = sc_ref[...]
    hw = lax.broadcasted_iota(jnp.int32, (_A, _HW), 1)
    sx = (hw % _W).astype(f32) * _STRIDE
    sy = (hw // _W).astype(f32) * _STRIDE
    wa = wa_ref[:, 0:1]
    ha = ha_ref[:, 0:1]
    ctr_x = cx_ref[:, 0:1] + sx
    ctr_y = cy_ref[:, 0:1] + sy
    h_im = im_ref[0]
    w_im = im_ref[1]
    scale = im_ref[2]
    pcx = dx_ref[...] * wa + ctr_x
    pcy = dy_ref[...] * ha + ctr_y
    pw = jnp.exp(jnp.minimum(dw_ref[...], _CLIP)) * wa
    ph = jnp.exp(jnp.minimum(dh_ref[...], _CLIP)) * ha
    x1 = pcx - 0.5 * pw
    y1 = pcy - 0.5 * ph
    x2 = pcx + 0.5 * pw - 1.0
    y2 = pcy + 0.5 * ph - 1.0
    x1 = jnp.clip(x1, 0.0, w_im - 1.0)
    y1 = jnp.clip(y1, 0.0, h_im - 1.0)
    x2 = jnp.clip(x2, 0.0, w_im - 1.0)
    y2 = jnp.clip(y2, 0.0, h_im - 1.0)
    ws = x2 - x1 + 1.0
    hs = y2 - y1 + 1.0
    xc = x1 + ws * 0.5
    yc = y1 + hs * 0.5
    msz = _MIN_SIZE * scale
    valid = (ws >= msz) & (hs >= msz) & (xc < w_im) & (yc < h_im)
    nscore = jnp.where(valid, score, -jnp.inf)

    bits = lax.bitcast_convert_type(score, jnp.int32)
    ikey = jnp.where(bits < 0, bits ^ jnp.int32(0x7FFFFFFF), bits)

    t_thr = jnp.int32(-(2 ** 31))
    for b in range(31, -1, -1):
        step_b = -(2 ** 31) if b == 31 else (1 << b)
        cand = t_thr + jnp.int32(step_b)
        cnt = jnp.sum((ikey >= cand).astype(jnp.int32))
        t_thr = jnp.where(cnt >= _PRE, cand, t_thr)

    c_gt = jnp.sum((ikey > t_thr).astype(jnp.int32))
    take_eq = jnp.int32(_PRE) - c_gt

    a_iota = lax.broadcasted_iota(jnp.int32, (_A, _HW), 0)
    iarr = hw * _A + a_iota
    eq = ikey == t_thr
    k_thr = jnp.int32(0)
    for b in range(17, -1, -1):
        cand = k_thr | jnp.int32(1 << b)
        cnt = jnp.sum((eq & (iarr < cand)).astype(jnp.int32))
        k_thr = jnp.where(cnt < take_eq, cand, k_thr)

    mask = (ikey > t_thr) | (eq & (iarr <= k_thr))
    maskf = mask.astype(f32)

    out_ref[0:_A, :] = x1
    out_ref[_A:2 * _A, :] = y1
    out_ref[2 * _A:3 * _A, :] = x2
    out_ref[3 * _A:4 * _A, :] = y2
    out_ref[4 * _A:5 * _A, :] = nscore
    out_ref[5 * _A:6 * _A, :] = maskf

    colsum = jnp.sum(maskf, axis=0, keepdims=True)
    hblk = lax.broadcasted_iota(jnp.int32, (_HW, _NW), 0) // _HW_PER_W
    jcol = lax.broadcasted_iota(jnp.int32, (_HW, _NW), 1)
    cum_sel = (hblk < jcol).astype(f32)
    offs = jnp.dot(colsum, cum_sel, preferred_element_type=f32)
    offs_ref[...] = jnp.zeros((8, 128), f32)
    offs_ref[0:1, 0:_NW] = offs


def _prep(score, dx, dy, dw, dh, wa, ha, cxa, cya, imv, interpret=False):
    vspec = pl.BlockSpec(memory_space=pltpu.VMEM)
    return pl.pallas_call(
        _prep_body,
        out_shape=(
            jax.ShapeDtypeStruct((6 * _A, _HW), jnp.float32),
            jax.ShapeDtypeStruct((8, 128), jnp.float32),
        ),
        in_specs=[vspec] * 9 + [pl.BlockSpec(memory_space=pltpu.SMEM)],
        out_specs=(vspec, vspec),
        interpret=interpret,
    )(score, dx, dy, dw, dh, wa, ha, cxa, cya, imv)


def _nms_body(x1_ref, y1_ref, x2_ref, y2_ref, sc_ref, id_ref, out_ref):
    f32 = jnp.float32
    x1 = x1_ref[...]
    y1 = y1_ref[...]
    x2 = x2_ref[...]
    y2 = y2_ref[...]
    jglob = (lax.broadcasted_iota(jnp.int32, (_ROWS, 128), 0) * 128
             + lax.broadcasted_iota(jnp.int32, (_ROWS, 128), 1))
    score = jnp.where(jglob < _PRE, sc_ref[...], -jnp.inf)
    idxf = id_ref[...]
    ws = x2 - x1 + 1.0
    hs = y2 - y1 + 1.0
    area = ws * hs
    lane = lax.broadcasted_iota(jnp.int32, (1, 128), 1)

    def step(t, avail):
        skey = jnp.where(avail > 0.0, score, -jnp.inf)
        m = jnp.max(skey)
        anyv = m > -jnp.inf
        selmask_s = skey == m
        isel = jnp.min(jnp.where(selmask_s, idxf, jnp.inf))
        selmask = selmask_s & (idxf == isel)
        bx1 = jnp.sum(jnp.where(selmask, x1, 0.0))
        by1 = jnp.sum(jnp.where(selmask, y1, 0.0))
        bx2 = jnp.sum(jnp.where(selmask, x2, 0.0))
        by2 = jnp.sum(jnp.where(selmask, y2, 0.0))
        barea = jnp.sum(jnp.where(selmask, area, 0.0))
        xx1 = jnp.maximum(x1, bx1)
        yy1 = jnp.maximum(y1, by1)
        xx2 = jnp.minimum(x2, bx2)
        yy2 = jnp.minimum(y2, by2)
        iw = jnp.maximum(xx2 - xx1 + 1.0, 0.0)
        ih = jnp.maximum(yy2 - yy1 + 1.0, 0.0)
        inter = iw * ih
        iou = inter / (barea + area - inter)
        kill = (iou > _THRESH) | selmask
        new_avail = jnp.where(anyv & kill, 0.0, avail)
        vals = jnp.where(lane == 0, bx1,
               jnp.where(lane == 1, by1,
               jnp.where(lane == 2, bx2,
               jnp.where(lane == 3, by2,
               jnp.where(lane == 4, m, 0.0)))))
        row = jnp.where(anyv, vals, jnp.zeros_like(vals))
        out_ref[pl.ds(t, 1), :] = row.astype(f32)
        return new_avail

    avail0 = jnp.ones((_ROWS, 128), jnp.float32)
    lax.fori_loop(0, _POST, step, avail0)


def _nms(x1, y1, x2, y2, sc, idf, interpret=False):
    vspec = pl.BlockSpec(memory_space=pltpu.VMEM)
    return pl.pallas_call(
        _nms_body,
        out_shape=jax.ShapeDtypeStruct((_POST, 128), jnp.float32),
        in_specs=[vspec] * 6,
        out_specs=vspec,
        interpret=interpret,
    )(x1, y1, x2, y2, sc, idf)


def _compact_xla(packed):
    maskf = packed[5 * _A:6 * _A].reshape(-1)
    mask = maskf > 0.5
    pos = jnp.cumsum(mask.astype(jnp.int32)) - 1
    tgt = jnp.where(mask, pos, _PAD)

    def comp(v):
        return jnp.zeros((_PAD,), jnp.float32).at[tgt].set(v, mode="drop")

    p = jnp.arange(_N_TOT)
    iorig = ((p % _HW) * _A + p // _HW).astype(jnp.float32)
    outs = [comp(packed[q * _A:(q + 1) * _A].reshape(-1)) for q in range(5)]
    outs.append(comp(iorig))
    return outs


def kernel(rpn_cls_prob, rpn_bbox_pred, im_info, anchors, _interpret=False):
    score = rpn_cls_prob.reshape(_A, _HW).astype(jnp.float32)
    d = rpn_bbox_pred.reshape(_A, 4, _HW).astype(jnp.float32)
    dx, dy, dw, dh = d[:, 0, :], d[:, 1, :], d[:, 2, :], d[:, 3, :]
    anchors = anchors.astype(jnp.float32)
    wa = anchors[:, 2] - anchors[:, 0] + 1.0
    ha = anchors[:, 3] - anchors[:, 1] + 1.0
    cxa = anchors[:, 0] + 0.5 * wa
    cya = anchors[:, 1] + 0.5 * ha
    bc = lambda v: jnp.broadcast_to(v[:, None], (_A, 128))
    imv = im_info.reshape(3).astype(jnp.float32)

    packed, _offs = _prep(score, dx, dy, dw, dh,
                          bc(wa), bc(ha), bc(cxa), bc(cya), imv,
                          interpret=_interpret)
    x1c, y1c, x2c, y2c, scc, idc = _compact_xla(packed)
    r = lambda v: v.reshape(_ROWS, 128)
    out = _nms(r(x1c), r(y1c), r(x2c), r(y2c), r(scc), r(idc),
               interpret=_interpret)
    rois = jnp.concatenate(
        [jnp.zeros((_POST, 1), jnp.float32), out[:, 0:4]], axis=1)
    probs = out[:, 4:5]
    return rois, probs
